# Initial kernel scaffold; baseline (speedup 1.0000x reference)
import jax
import jax.numpy as jnp
from jax.experimental import pallas as pl


def kernel(entity, edge_index_all, embedding, W, b):
    raise NotImplementedError("write your pallas kernel here")


# speedup vs baseline: 16.3242x; 16.3242x over previous
import functools

import jax
import jax.numpy as jnp
from jax import lax
from jax.experimental import pallas as pl
from jax.experimental.pallas import tpu as pltpu
from jax.experimental.pallas import tpu_sc as plsc

N = 50000
E = 800000
V = 100000
D = 100
DP = 128
NP = 50176
CH = 12544
CPAD = CH + 16
ET = E // 16
SB = 10000
NVB = SB // 16


def _rsqrt16(d):
    xi = plsc.bitcast(d, jnp.int32)
    yi = jnp.int32(0x5F3759DF) - lax.shift_right_logical(xi, 1)
    y = plsc.bitcast(yi, jnp.float32)
    for _ in range(3):
        y = y * (1.5 - 0.5 * d * y * y)
    return jnp.where(d == 0.0, jnp.float32(jnp.inf), y)


def _k1_body(ent_ref, dst_ref, emb_ref, x_out, dis_out,
             hist_all, hist_v, dstbuf, hbuf, degbuf, disbuf, idxb, rowsb,
             sem):
    c = lax.axis_index("c")
    s = lax.axis_index("s")

    @pl.when(c == 1)
    def _gather_x():
        for k in range(28):
            base = s * 3136 + k * 112
            pltpu.sync_copy(ent_ref.at[pl.ds(base, 112)], idxb)
            pltpu.async_copy(emb_ref.at[idxb], rowsb, sem).wait()
            pltpu.sync_copy(rowsb, x_out.at[pl.ds(base, 112)])

    @pl.when(c == 0)
    def _hist_and_dis():
        zero = jnp.zeros((16,), jnp.float32)

        @pl.when(s < 8)
        def _build_hist():
            def _z(i, _):
                hist_v[pl.ds(i * 16, 16)] = zero
                return 0
            lax.fori_loop(0, NP // 16, _z, 0)

            ones = jnp.ones((16,), jnp.float32)
            for blk in range(2 * ET // SB):
                pltpu.sync_copy(
                    dst_ref.at[pl.ds(s * 2 * ET + blk * SB, SB)], dstbuf)

                def _acc(i, _):
                    dv = dstbuf[pl.ds(i * 16, 16)]
                    plsc.addupdate_scatter(hist_v, [dv], ones)
                    return 0
                lax.fori_loop(0, NVB, _acc, 0)

            pltpu.sync_copy(hist_v, hist_all.at[pl.ds(s * NP, NP)])

        plsc.subcore_barrier()
        def _zd(i, _):
            degbuf[pl.ds(i * 16, 16)] = zero
            return 0
        lax.fori_loop(0, 196, _zd, 0)
        for t in range(8):
            pltpu.sync_copy(hist_all.at[pl.ds(t * NP + s * 3136, 3136)], hbuf)

            def _sum(i, _):
                degbuf[pl.ds(i * 16, 16)] = (
                    degbuf[pl.ds(i * 16, 16)] + hbuf[pl.ds(i * 16, 16)])
                return 0
            lax.fori_loop(0, 196, _sum, 0)

        def _dis(i, _):
            disbuf[pl.ds(i * 16, 16)] = _rsqrt16(degbuf[pl.ds(i * 16, 16)])
            return 0
        lax.fori_loop(0, 196, _dis, 0)
        pltpu.sync_copy(disbuf, dis_out.at[pl.ds(s * 3136, 3136)])


_k1 = functools.partial(
    pl.kernel,
    mesh=plsc.VectorSubcoreMesh(core_axis_name="c", subcore_axis_name="s"),
    compiler_params=pltpu.CompilerParams(needs_layout_passes=False),
    out_type=[
        jax.ShapeDtypeStruct((NP, DP), jnp.float32),
        jax.ShapeDtypeStruct((NP,), jnp.float32),
    ],
    scratch_types=[
        pltpu.VMEM_SHARED((8 * NP,), jnp.float32),
        pltpu.VMEM((NP,), jnp.float32),
        pltpu.VMEM((SB,), jnp.int32),
        pltpu.VMEM((3136,), jnp.float32),
        pltpu.VMEM((3136,), jnp.float32),
        pltpu.VMEM((3136,), jnp.float32),
        pltpu.VMEM((112,), jnp.int32),
        pltpu.VMEM((112, DP), jnp.float32),
        pltpu.SemaphoreType.DMA,
    ],
)(_k1_body)


SB2 = 2000
FK = 112
STCAP = 2304


def _k3_body(src_ref, dst_ref, xs_ref, agg_out,
             accum_sh, stage, srcb, dstb, rowsb, idxb, dlb, sem):
    c = lax.axis_index("c")
    s = lax.axis_index("s")

    zero = jnp.zeros((16,), jnp.float32)
    iota = lax.iota(jnp.int32, 16)
    dummy = (s * 16 + iota) | lax.shift_left(CH + iota, 17)

    def _fire(g, _):
        for j in range(FK // 16):
            v = stage[pl.ds(g * FK + j * 16, 16)]
            idxb[pl.ds(j * 16, 16)] = v & jnp.int32(0x1FFFF)
            dlb[pl.ds(j * 16, 16)] = lax.shift_right_logical(v, 17)
        pltpu.async_copy(xs_ref.at[idxb], rowsb, sem).wait()
        pltpu.sync_copy(rowsb, accum_sh.at[dlb], add=True)
        return 0

    for p in range(2):
        chunk = 2 * p + c
        r0 = chunk * CH

        def _zz(i, _):
            for j in range(DP // 16):
                rowsb[i, pl.ds(j * 16, 16)] = zero
            return 0
        lax.fori_loop(0, FK, _zz, 0)
        for k in range(7):
            pltpu.sync_copy(rowsb, accum_sh.at[pl.ds(s * 784 + k * 112, 112)])
        plsc.subcore_barrier()

        def _scan_blk(blk, cnt):
            pltpu.sync_copy(src_ref.at[pl.ds(s * ET + blk * SB2, SB2)], srcb)
            pltpu.sync_copy(dst_ref.at[pl.ds(s * ET + blk * SB2, SB2)], dstb)

            def _scan(i, cn):
                sv = srcb[pl.ds(i * 16, 16)]
                dv = dstb[pl.ds(i * 16, 16)]
                m = (dv >= r0) & (dv < r0 + CH)
                packed = sv | lax.shift_left(dv - r0, 17)
                plsc.store_compressed(stage.at[pl.ds(cn, 16)], packed, mask=m)
                return cn + jnp.sum(m.astype(jnp.int32))
            cnt = lax.fori_loop(0, SB2 // 16, _scan, cnt)

            ng = cnt // FK
            lax.fori_loop(0, ng, _fire, 0)
            base = ng * FK
            for j in range(FK // 16):
                v = stage[pl.ds(base + j * 16, 16)]
                stage[pl.ds(j * 16, 16)] = v
            return cnt - base

        cnt = lax.fori_loop(0, ET // SB2, _scan_blk, jnp.int32(0))

        for k in range(FK // 16):
            stage[pl.ds(cnt + k * 16, 16)] = dummy
        lax.fori_loop(0, (cnt + FK - 1) // FK, _fire, 0)
        plsc.subcore_barrier()

        for k in range(7):
            pltpu.sync_copy(accum_sh.at[pl.ds(s * 784 + k * 112, 112)],
                            agg_out.at[pl.ds(r0 + s * 784 + k * 112, 112)])
        plsc.subcore_barrier()


_k3 = functools.partial(
    pl.kernel,
    mesh=plsc.VectorSubcoreMesh(core_axis_name="c", subcore_axis_name="s"),
    compiler_params=pltpu.CompilerParams(needs_layout_passes=False),
    out_type=jax.ShapeDtypeStruct((NP, DP), jnp.float32),
    scratch_types=[
        pltpu.VMEM_SHARED((CPAD, DP), jnp.float32),
        pltpu.VMEM((STCAP,), jnp.int32),
        pltpu.VMEM((SB2,), jnp.int32),
        pltpu.VMEM((SB2,), jnp.int32),
        pltpu.VMEM((FK, DP), jnp.float32),
        pltpu.VMEM((FK,), jnp.int32),
        pltpu.VMEM((FK,), jnp.int32),
        pltpu.SemaphoreType.DMA,
    ],
)(_k3_body)


def _k2_body(x_ref, d_ref, wt_ref, b_ref, gate_ref, xs_ref):
    x = x_ref[...]
    g = jnp.dot(x, wt_ref[...], preferred_element_type=jnp.float32)
    gate_ref[...] = jax.nn.sigmoid(g + b_ref[...])
    xs_ref[...] = x * d_ref[...]


RB = 512


def _k2(x_pad, dis2, wt_pad, b2):
    return pl.pallas_call(
        _k2_body,
        grid=(NP // RB,),
        in_specs=[
            pl.BlockSpec((RB, DP), lambda i: (i, 0)),
            pl.BlockSpec((RB, 1), lambda i: (i, 0)),
            pl.BlockSpec((DP, DP), lambda i: (0, 0)),
            pl.BlockSpec((1, DP), lambda i: (0, 0)),
        ],
        out_specs=[
            pl.BlockSpec((RB, DP), lambda i: (i, 0)),
            pl.BlockSpec((RB, DP), lambda i: (i, 0)),
        ],
        out_shape=[
            jax.ShapeDtypeStruct((NP, DP), jnp.float32),
            jax.ShapeDtypeStruct((NP, DP), jnp.float32),
        ],
    )(x_pad, dis2, wt_pad, b2)


def _k4_body(s_ref, d_ref, g_ref, x_ref, o_ref):
    d = d_ref[...]
    dsafe = jnp.where(jnp.isinf(d), jnp.float32(0.0), d)
    agg = jnp.maximum(dsafe * s_ref[...], 0.0)
    g = g_ref[...]
    o_ref[...] = g * agg + (1.0 - g) * x_ref[...]


def _k4(agg, dis2, gate, x_pad):
    return pl.pallas_call(
        _k4_body,
        grid=(NP // RB,),
        in_specs=[
            pl.BlockSpec((RB, DP), lambda i: (i, 0)),
            pl.BlockSpec((RB, 1), lambda i: (i, 0)),
            pl.BlockSpec((RB, DP), lambda i: (i, 0)),
            pl.BlockSpec((RB, DP), lambda i: (i, 0)),
        ],
        out_specs=pl.BlockSpec((RB, DP), lambda i: (i, 0)),
        out_shape=jax.ShapeDtypeStruct((NP, DP), jnp.float32),
    )(agg, dis2, gate, x_pad)


def kernel(entity, edge_index_all, embedding, W, b):
    emb_pad = jnp.pad(embedding, ((0, 0), (0, DP - D)))
    ent_pad = jnp.pad(entity, (0, NP - N))
    src = edge_index_all[0]
    dst = edge_index_all[1]
    wt_pad = jnp.pad(W.T, ((0, DP - D), (0, DP - D)))
    b2 = jnp.pad(b, (0, DP - D)).reshape(1, DP)

    x_pad, dis = _k1(ent_pad, dst, emb_pad)
    dis2 = dis.reshape(NP, 1)
    gate, xs = _k2(x_pad, dis2, wt_pad, b2)
    agg = _k3(src, dst, xs)
    out_pad = _k4(agg, dis2, gate, x_pad)
    return out_pad[:N, :D]


# speedup vs baseline: 17.0329x; 1.0434x over previous
import functools

import jax
import jax.numpy as jnp
from jax import lax
from jax.experimental import pallas as pl
from jax.experimental.pallas import tpu as pltpu
from jax.experimental.pallas import tpu_sc as plsc

N = 50000
E = 800000
V = 100000
D = 100
DP = 128
NP = 50176
CH = 12544
CPAD = CH + 16
ET = E // 16
SB = 10000
NVB = SB // 16


def _rsqrt16(d):
    xi = plsc.bitcast(d, jnp.int32)
    yi = jnp.int32(0x5F3759DF) - lax.shift_right_logical(xi, 1)
    y = plsc.bitcast(yi, jnp.float32)
    for _ in range(3):
        y = y * (1.5 - 0.5 * d * y * y)
    return jnp.where(d == 0.0, jnp.float32(jnp.inf), y)


def _k1_body(ent_ref, dst_ref, emb_ref, x_out, dis_out,
             hist_all, hist_v, dstbuf, hbuf, degbuf, disbuf, idxb, rowsb,
             sem):
    c = lax.axis_index("c")
    s = lax.axis_index("s")

    @pl.when(c == 1)
    def _gather_x():
        for k in range(28):
            base = s * 3136 + k * 112
            pltpu.sync_copy(ent_ref.at[pl.ds(base, 112)], idxb)
            pltpu.async_copy(emb_ref.at[idxb], rowsb, sem).wait()
            pltpu.sync_copy(rowsb, x_out.at[pl.ds(base, 112)])

    @pl.when(c == 0)
    def _hist_and_dis():
        zero = jnp.zeros((16,), jnp.float32)

        @pl.when(s < 8)
        def _build_hist():
            def _z(i, _):
                hist_v[pl.ds(i * 16, 16)] = zero
                return 0
            lax.fori_loop(0, NP // 16, _z, 0)

            ones = jnp.ones((16,), jnp.float32)
            for blk in range(2 * ET // SB):
                pltpu.sync_copy(
                    dst_ref.at[pl.ds(s * 2 * ET + blk * SB, SB)], dstbuf)

                def _acc(i, _):
                    dv = dstbuf[pl.ds(i * 16, 16)]
                    plsc.addupdate_scatter(hist_v, [dv], ones)
                    return 0
                lax.fori_loop(0, NVB, _acc, 0)

            pltpu.sync_copy(hist_v, hist_all.at[pl.ds(s * NP, NP)])

        plsc.subcore_barrier()
        def _zd(i, _):
            degbuf[pl.ds(i * 16, 16)] = zero
            return 0
        lax.fori_loop(0, 196, _zd, 0)
        for t in range(8):
            pltpu.sync_copy(hist_all.at[pl.ds(t * NP + s * 3136, 3136)], hbuf)

            def _sum(i, _):
                degbuf[pl.ds(i * 16, 16)] = (
                    degbuf[pl.ds(i * 16, 16)] + hbuf[pl.ds(i * 16, 16)])
                return 0
            lax.fori_loop(0, 196, _sum, 0)

        def _dis(i, _):
            disbuf[pl.ds(i * 16, 16)] = _rsqrt16(degbuf[pl.ds(i * 16, 16)])
            return 0
        lax.fori_loop(0, 196, _dis, 0)
        pltpu.sync_copy(disbuf, dis_out.at[pl.ds(s * 3136, 3136)])


_k1 = functools.partial(
    pl.kernel,
    mesh=plsc.VectorSubcoreMesh(core_axis_name="c", subcore_axis_name="s"),
    compiler_params=pltpu.CompilerParams(needs_layout_passes=False),
    out_type=[
        jax.ShapeDtypeStruct((NP, DP), jnp.float32),
        jax.ShapeDtypeStruct((NP,), jnp.float32),
    ],
    scratch_types=[
        pltpu.VMEM_SHARED((8 * NP,), jnp.float32),
        pltpu.VMEM((NP,), jnp.float32),
        pltpu.VMEM((SB,), jnp.int32),
        pltpu.VMEM((3136,), jnp.float32),
        pltpu.VMEM((3136,), jnp.float32),
        pltpu.VMEM((3136,), jnp.float32),
        pltpu.VMEM((112,), jnp.int32),
        pltpu.VMEM((112, DP), jnp.float32),
        pltpu.SemaphoreType.DMA,
    ],
)(_k1_body)


SB2 = 2000
FK = 80
STCAP = 2096


def _k3_body(src_ref, dst_ref, xs_ref, agg_out,
             accum_sh, stage, srcb, dstb, rowsb0, rowsb1,
             idxb0, idxb1, dlb0, dlb1, sem0, sem1):
    c = lax.axis_index("c")
    s = lax.axis_index("s")

    zero = jnp.zeros((16,), jnp.float32)
    iota = lax.iota(jnp.int32, 16)
    dummy = (s * 16 + iota) | lax.shift_left(CH + iota, 17)

    def _unpack(g, idxb, dlb):
        for j in range(FK // 16):
            v = stage[pl.ds(g * FK + j * 16, 16)]
            idxb[pl.ds(j * 16, 16)] = v & jnp.int32(0x1FFFF)
            dlb[pl.ds(j * 16, 16)] = lax.shift_right_logical(v, 17)

    def _fire_single(g, _):
        _unpack(g, idxb0, dlb0)
        pltpu.async_copy(xs_ref.at[idxb0], rowsb0, sem0).wait()
        pltpu.sync_copy(rowsb0, accum_sh.at[dlb0], add=True)
        return 0

    for p in range(2):
        chunk = 2 * p + c
        r0 = chunk * CH

        def _zz(i, _):
            for j in range(DP // 16):
                rowsb0[i, pl.ds(j * 16, 16)] = zero
            return 0
        lax.fori_loop(0, FK, _zz, 0)
        for k in range(9):
            pltpu.sync_copy(rowsb0.at[pl.ds(0, 80)],
                            accum_sh.at[pl.ds(s * 784 + k * 80, 80)])
        pltpu.sync_copy(rowsb0.at[pl.ds(0, 64)],
                        accum_sh.at[pl.ds(s * 784 + 720, 64)])
        plsc.subcore_barrier()

        def _scan_blk(blk, cnt):
            pltpu.sync_copy(src_ref.at[pl.ds(s * ET + blk * SB2, SB2)], srcb)
            pltpu.sync_copy(dst_ref.at[pl.ds(s * ET + blk * SB2, SB2)], dstb)

            def _scan(i, cn):
                sv = srcb[pl.ds(i * 16, 16)]
                dv = dstb[pl.ds(i * 16, 16)]
                m = (dv >= r0) & (dv < r0 + CH)
                packed = sv | lax.shift_left(dv - r0, 17)
                plsc.store_compressed(stage.at[pl.ds(cn, 16)], packed, mask=m)
                return cn + jnp.sum(m.astype(jnp.int32))
            cnt = lax.fori_loop(0, SB2 // 16, _scan, cnt)

            ng = cnt // FK

            def _fire_pair(k, _):
                g0 = 2 * k
                g1 = g0 + 1
                _unpack(g0, idxb0, dlb0)
                cp0 = pltpu.async_copy(xs_ref.at[idxb0], rowsb0, sem0)

                @pl.when(g1 < ng)
                def _issue1():
                    _unpack(g1, idxb1, dlb1)
                    pltpu.async_copy(xs_ref.at[idxb1], rowsb1, sem1)

                cp0.wait()
                pltpu.sync_copy(rowsb0, accum_sh.at[dlb0], add=True)

                @pl.when(g1 < ng)
                def _drain1():
                    pltpu.make_async_copy(xs_ref.at[idxb1], rowsb1,
                                          sem1).wait()
                    pltpu.sync_copy(rowsb1, accum_sh.at[dlb1], add=True)
                return 0
            lax.fori_loop(0, (ng + 1) // 2, _fire_pair, 0)

            base = ng * FK
            for j in range(FK // 16):
                v = stage[pl.ds(base + j * 16, 16)]
                stage[pl.ds(j * 16, 16)] = v
            return cnt - base

        cnt = lax.fori_loop(0, ET // SB2, _scan_blk, jnp.int32(0))

        for k in range(FK // 16):
            stage[pl.ds(cnt + k * 16, 16)] = dummy
        lax.fori_loop(0, (cnt + FK - 1) // FK, _fire_single, 0)
        plsc.subcore_barrier()

        for k in range(7):
            pltpu.sync_copy(accum_sh.at[pl.ds(s * 784 + k * 112, 112)],
                            agg_out.at[pl.ds(r0 + s * 784 + k * 112, 112)])
        plsc.subcore_barrier()


_k3 = functools.partial(
    pl.kernel,
    mesh=plsc.VectorSubcoreMesh(core_axis_name="c", subcore_axis_name="s"),
    compiler_params=pltpu.CompilerParams(needs_layout_passes=False),
    out_type=jax.ShapeDtypeStruct((NP, DP), jnp.float32),
    scratch_types=[
        pltpu.VMEM_SHARED((CPAD, DP), jnp.float32),
        pltpu.VMEM((STCAP,), jnp.int32),
        pltpu.VMEM((SB2,), jnp.int32),
        pltpu.VMEM((SB2,), jnp.int32),
        pltpu.VMEM((FK, DP), jnp.float32),
        pltpu.VMEM((FK, DP), jnp.float32),
        pltpu.VMEM((FK,), jnp.int32),
        pltpu.VMEM((FK,), jnp.int32),
        pltpu.VMEM((FK,), jnp.int32),
        pltpu.VMEM((FK,), jnp.int32),
        pltpu.SemaphoreType.DMA,
        pltpu.SemaphoreType.DMA,
    ],
)(_k3_body)


def _k2_body(x_ref, d_ref, wt_ref, b_ref, gate_ref, xs_ref):
    x = x_ref[...]
    g = jnp.dot(x, wt_ref[...], preferred_element_type=jnp.float32)
    gate_ref[...] = jax.nn.sigmoid(g + b_ref[...])
    xs_ref[...] = x * d_ref[...]


RB = 512


def _k2(x_pad, dis2, wt_pad, b2):
    return pl.pallas_call(
        _k2_body,
        grid=(NP // RB,),
        in_specs=[
            pl.BlockSpec((RB, DP), lambda i: (i, 0)),
            pl.BlockSpec((RB, 1), lambda i: (i, 0)),
            pl.BlockSpec((DP, DP), lambda i: (0, 0)),
            pl.BlockSpec((1, DP), lambda i: (0, 0)),
        ],
        out_specs=[
            pl.BlockSpec((RB, DP), lambda i: (i, 0)),
            pl.BlockSpec((RB, DP), lambda i: (i, 0)),
        ],
        out_shape=[
            jax.ShapeDtypeStruct((NP, DP), jnp.float32),
            jax.ShapeDtypeStruct((NP, DP), jnp.float32),
        ],
    )(x_pad, dis2, wt_pad, b2)


def _k4_body(s_ref, d_ref, g_ref, x_ref, o_ref):
    d = d_ref[...]
    dsafe = jnp.where(jnp.isinf(d), jnp.float32(0.0), d)
    agg = jnp.maximum(dsafe * s_ref[...], 0.0)
    g = g_ref[...]
    o_ref[...] = g * agg + (1.0 - g) * x_ref[...]


def _k4(agg, dis2, gate, x_pad):
    return pl.pallas_call(
        _k4_body,
        grid=(NP // RB,),
        in_specs=[
            pl.BlockSpec((RB, DP), lambda i: (i, 0)),
            pl.BlockSpec((RB, 1), lambda i: (i, 0)),
            pl.BlockSpec((RB, DP), lambda i: (i, 0)),
            pl.BlockSpec((RB, DP), lambda i: (i, 0)),
        ],
        out_specs=pl.BlockSpec((RB, DP), lambda i: (i, 0)),
        out_shape=jax.ShapeDtypeStruct((NP, DP), jnp.float32),
    )(agg, dis2, gate, x_pad)


def kernel(entity, edge_index_all, embedding, W, b):
    emb_pad = jnp.pad(embedding, ((0, 0), (0, DP - D)))
    ent_pad = jnp.pad(entity, (0, NP - N))
    src = edge_index_all[0]
    dst = edge_index_all[1]
    wt_pad = jnp.pad(W.T, ((0, DP - D), (0, DP - D)))
    b2 = jnp.pad(b, (0, DP - D)).reshape(1, DP)

    x_pad, dis = _k1(ent_pad, dst, emb_pad)
    dis2 = dis.reshape(NP, 1)
    gate, xs = _k2(x_pad, dis2, wt_pad, b2)
    agg = _k3(src, dst, xs)
    out_pad = _k4(agg, dis2, gate, x_pad)
    return out_pad[:N, :D]


# speedup vs baseline: 17.6248x; 1.0347x over previous
import functools

import jax
import jax.numpy as jnp
from jax import lax
from jax.experimental import pallas as pl
from jax.experimental.pallas import tpu as pltpu
from jax.experimental.pallas import tpu_sc as plsc

N = 50000
E = 800000
V = 100000
D = 100
DP = 128
NP = 50176
CH = 12544
CPAD = CH + 16
ET = E // 16
SB = 10000
NVB = SB // 16


def _rsqrt16(d):
    xi = plsc.bitcast(d, jnp.int32)
    yi = jnp.int32(0x5F3759DF) - lax.shift_right_logical(xi, 1)
    y = plsc.bitcast(yi, jnp.float32)
    for _ in range(3):
        y = y * (1.5 - 0.5 * d * y * y)
    return jnp.where(d == 0.0, jnp.float32(jnp.inf), y)


def _k1_body(ent_ref, dst_ref, emb_ref, x_out, xs_out, dis_out,
             hist_all, dis_sh, hist_v, dstbuf, hbuf, degbuf, disbuf, disc,
             idxa, idxb, rowsa, rowsb, sem0, sem1):
    c = lax.axis_index("c")
    s = lax.axis_index("s")
    w = c * 16 + s
    zero = jnp.zeros((16,), jnp.float32)

    @pl.when(s < 8)
    def _build_hist():
        def _z(i, _):
            hist_v[pl.ds(i * 16, 16)] = zero
            return 0
        lax.fori_loop(0, NP // 16, _z, 0)

        ones = jnp.ones((16,), jnp.float32)
        for blk in range(2 * ET // SB):
            pltpu.sync_copy(
                dst_ref.at[pl.ds(s * 2 * ET + blk * SB, SB)], dstbuf)

            def _acc(i, _):
                dv = dstbuf[pl.ds(i * 16, 16)]
                plsc.addupdate_scatter(hist_v, [dv], ones)
                return 0
            lax.fori_loop(0, NVB, _acc, 0)

        pltpu.sync_copy(hist_v, hist_all.at[pl.ds(s * NP, NP)])

    plsc.subcore_barrier()

    def _zd(i, _):
        degbuf[pl.ds(i * 16, 16)] = zero
        return 0
    lax.fori_loop(0, 196, _zd, 0)
    for t in range(8):
        pltpu.sync_copy(hist_all.at[pl.ds(t * NP + s * 3136, 3136)], hbuf)

        def _sum(i, _):
            degbuf[pl.ds(i * 16, 16)] = (
                degbuf[pl.ds(i * 16, 16)] + hbuf[pl.ds(i * 16, 16)])
            return 0
        lax.fori_loop(0, 196, _sum, 0)

    def _dis(i, _):
        disbuf[pl.ds(i * 16, 16)] = _rsqrt16(degbuf[pl.ds(i * 16, 16)])
        return 0
    lax.fori_loop(0, 196, _dis, 0)
    pltpu.sync_copy(disbuf, dis_sh.at[pl.ds(s * 3136, 3136)])

    @pl.when(c == 0)
    def _dis_hbm():
        pltpu.sync_copy(disbuf, dis_out.at[pl.ds(s * 3136, 3136)])

    plsc.subcore_barrier()

    pltpu.sync_copy(dis_sh.at[pl.ds(w * 1568, 1568)], disc)
    bufs = [(idxa, rowsa, sem0), (idxb, rowsb, sem1)]

    def _issue(k):
        idx, rows, sem = bufs[k % 2]
        base = w * 1568 + k * 112
        pltpu.sync_copy(ent_ref.at[pl.ds(base, 112)], idx)
        pltpu.async_copy(emb_ref.at[idx], rows, sem)

    _issue(0)
    for k in range(14):
        idx, rows, sem = bufs[k % 2]
        base = w * 1568 + k * 112
        pltpu.make_async_copy(emb_ref.at[idx], rows, sem).wait()
        if k + 1 < 14:
            _issue(k + 1)
        pltpu.sync_copy(rows, x_out.at[pl.ds(base, 112)])

        def _scale(r, _):
            sp = plsc.load_gather(
                disc, [jnp.full((16,), k * 112 + r, jnp.int32)])
            for j in range(DP // 16):
                rows[r, pl.ds(j * 16, 16)] = rows[r, pl.ds(j * 16, 16)] * sp
            return 0
        lax.fori_loop(0, 112, _scale, 0)
        pltpu.sync_copy(rows, xs_out.at[pl.ds(base, 112)])


_k1 = functools.partial(
    pl.kernel,
    mesh=plsc.VectorSubcoreMesh(core_axis_name="c", subcore_axis_name="s"),
    compiler_params=pltpu.CompilerParams(needs_layout_passes=False),
    out_type=[
        jax.ShapeDtypeStruct((NP, DP), jnp.float32),
        jax.ShapeDtypeStruct((NP, DP), jnp.float32),
        jax.ShapeDtypeStruct((NP,), jnp.float32),
    ],
    scratch_types=[
        pltpu.VMEM_SHARED((8 * NP,), jnp.float32),
        pltpu.VMEM_SHARED((NP,), jnp.float32),
        pltpu.VMEM((NP,), jnp.float32),
        pltpu.VMEM((SB,), jnp.int32),
        pltpu.VMEM((3136,), jnp.float32),
        pltpu.VMEM((3136,), jnp.float32),
        pltpu.VMEM((3136,), jnp.float32),
        pltpu.VMEM((1568,), jnp.float32),
        pltpu.VMEM((112,), jnp.int32),
        pltpu.VMEM((112,), jnp.int32),
        pltpu.VMEM((112, DP), jnp.float32),
        pltpu.VMEM((112, DP), jnp.float32),
        pltpu.SemaphoreType.DMA,
        pltpu.SemaphoreType.DMA,
    ],
)(_k1_body)


SB2 = 2000
FK = 80
STCAP = 2096


def _k3_body(src_ref, dst_ref, xs_ref, agg_out,
             accum_sh, stage, srcb, dstb, rowsb0, rowsb1,
             idxb0, idxb1, dlb0, dlb1, sem0, sem1):
    c = lax.axis_index("c")
    s = lax.axis_index("s")

    zero = jnp.zeros((16,), jnp.float32)
    iota = lax.iota(jnp.int32, 16)
    dummy = (s * 16 + iota) | lax.shift_left(CH + iota, 17)

    def _unpack(g, idxb, dlb):
        for j in range(FK // 16):
            v = stage[pl.ds(g * FK + j * 16, 16)]
            idxb[pl.ds(j * 16, 16)] = v & jnp.int32(0x1FFFF)
            dlb[pl.ds(j * 16, 16)] = lax.shift_right_logical(v, 17)

    def _fire_single(g, _):
        _unpack(g, idxb0, dlb0)
        pltpu.async_copy(xs_ref.at[idxb0], rowsb0, sem0).wait()
        pltpu.sync_copy(rowsb0, accum_sh.at[dlb0], add=True)
        return 0

    for p in range(2):
        chunk = 2 * p + c
        r0 = chunk * CH

        def _zz(i, _):
            for j in range(DP // 16):
                rowsb0[i, pl.ds(j * 16, 16)] = zero
            return 0
        lax.fori_loop(0, FK, _zz, 0)
        for k in range(9):
            pltpu.sync_copy(rowsb0.at[pl.ds(0, 80)],
                            accum_sh.at[pl.ds(s * 784 + k * 80, 80)])
        pltpu.sync_copy(rowsb0.at[pl.ds(0, 64)],
                        accum_sh.at[pl.ds(s * 784 + 720, 64)])
        plsc.subcore_barrier()

        def _scan_blk(blk, cnt):
            pltpu.sync_copy(src_ref.at[pl.ds(s * ET + blk * SB2, SB2)], srcb)
            pltpu.sync_copy(dst_ref.at[pl.ds(s * ET + blk * SB2, SB2)], dstb)

            def _scan(i, cn):
                sv = srcb[pl.ds(i * 16, 16)]
                dv = dstb[pl.ds(i * 16, 16)]
                m = (dv >= r0) & (dv < r0 + CH)
                packed = sv | lax.shift_left(dv - r0, 17)
                plsc.store_compressed(stage.at[pl.ds(cn, 16)], packed, mask=m)
                return cn + jnp.sum(m.astype(jnp.int32))
            cnt = lax.fori_loop(0, SB2 // 16, _scan, cnt)

            ng = cnt // FK

            def _fire_pair(k, _):
                g0 = 2 * k
                g1 = g0 + 1
                _unpack(g0, idxb0, dlb0)
                cp0 = pltpu.async_copy(xs_ref.at[idxb0], rowsb0, sem0)

                @pl.when(g1 < ng)
                def _issue1():
                    _unpack(g1, idxb1, dlb1)
                    pltpu.async_copy(xs_ref.at[idxb1], rowsb1, sem1)

                cp0.wait()
                pltpu.sync_copy(rowsb0, accum_sh.at[dlb0], add=True)

                @pl.when(g1 < ng)
                def _drain1():
                    pltpu.make_async_copy(xs_ref.at[idxb1], rowsb1,
                                          sem1).wait()
                    pltpu.sync_copy(rowsb1, accum_sh.at[dlb1], add=True)
                return 0
            lax.fori_loop(0, (ng + 1) // 2, _fire_pair, 0)

            base = ng * FK
            for j in range(FK // 16):
                v = stage[pl.ds(base + j * 16, 16)]
                stage[pl.ds(j * 16, 16)] = v
            return cnt - base

        cnt = lax.fori_loop(0, ET // SB2, _scan_blk, jnp.int32(0))

        for k in range(FK // 16):
            stage[pl.ds(cnt + k * 16, 16)] = dummy
        lax.fori_loop(0, (cnt + FK - 1) // FK, _fire_single, 0)
        plsc.subcore_barrier()

        for k in range(7):
            pltpu.sync_copy(accum_sh.at[pl.ds(s * 784 + k * 112, 112)],
                            agg_out.at[pl.ds(r0 + s * 784 + k * 112, 112)])
        plsc.subcore_barrier()


_k3 = functools.partial(
    pl.kernel,
    mesh=plsc.VectorSubcoreMesh(core_axis_name="c", subcore_axis_name="s"),
    compiler_params=pltpu.CompilerParams(needs_layout_passes=False),
    out_type=jax.ShapeDtypeStruct((NP, DP), jnp.float32),
    scratch_types=[
        pltpu.VMEM_SHARED((CPAD, DP), jnp.float32),
        pltpu.VMEM((STCAP,), jnp.int32),
        pltpu.VMEM((SB2,), jnp.int32),
        pltpu.VMEM((SB2,), jnp.int32),
        pltpu.VMEM((FK, DP), jnp.float32),
        pltpu.VMEM((FK, DP), jnp.float32),
        pltpu.VMEM((FK,), jnp.int32),
        pltpu.VMEM((FK,), jnp.int32),
        pltpu.VMEM((FK,), jnp.int32),
        pltpu.VMEM((FK,), jnp.int32),
        pltpu.SemaphoreType.DMA,
        pltpu.SemaphoreType.DMA,
    ],
)(_k3_body)


def _k2_body(x_ref, wt_ref, b_ref, gate_ref):
    x = x_ref[...]
    g = jnp.dot(x, wt_ref[...], preferred_element_type=jnp.float32)
    gate_ref[...] = jax.nn.sigmoid(g + b_ref[...])


RB = 512


def _k2(x_pad, wt_pad, b2):
    return pl.pallas_call(
        _k2_body,
        grid=(NP // RB,),
        in_specs=[
            pl.BlockSpec((RB, DP), lambda i: (i, 0)),
            pl.BlockSpec((DP, DP), lambda i: (0, 0)),
            pl.BlockSpec((1, DP), lambda i: (0, 0)),
        ],
        out_specs=pl.BlockSpec((RB, DP), lambda i: (i, 0)),
        out_shape=jax.ShapeDtypeStruct((NP, DP), jnp.float32),
    )(x_pad, wt_pad, b2)


def _k4_body(s_ref, d_ref, g_ref, x_ref, o_ref):
    d = d_ref[...]
    dsafe = jnp.where(jnp.isinf(d), jnp.float32(0.0), d)
    agg = jnp.maximum(dsafe * s_ref[...], 0.0)
    g = g_ref[...]
    o_ref[...] = g * agg + (1.0 - g) * x_ref[...]


def _k4(agg, dis2, gate, x_pad):
    return pl.pallas_call(
        _k4_body,
        grid=(NP // RB,),
        in_specs=[
            pl.BlockSpec((RB, DP), lambda i: (i, 0)),
            pl.BlockSpec((RB, 1), lambda i: (i, 0)),
            pl.BlockSpec((RB, DP), lambda i: (i, 0)),
            pl.BlockSpec((RB, DP), lambda i: (i, 0)),
        ],
        out_specs=pl.BlockSpec((RB, DP), lambda i: (i, 0)),
        out_shape=jax.ShapeDtypeStruct((NP, DP), jnp.float32),
    )(agg, dis2, gate, x_pad)


def kernel(entity, edge_index_all, embedding, W, b):
    emb_pad = jnp.pad(embedding, ((0, 0), (0, DP - D)))
    ent_pad = jnp.pad(entity, (0, NP - N))
    src = edge_index_all[0]
    dst = edge_index_all[1]
    wt_pad = jnp.pad(W.T, ((0, DP - D), (0, DP - D)))
    b2 = jnp.pad(b, (0, DP - D)).reshape(1, DP)

    x_pad, xs, dis = _k1(ent_pad, dst, emb_pad)
    dis2 = dis.reshape(NP, 1)
    agg = _k3(src, dst, xs)
    gate = _k2(x_pad, wt_pad, b2)
    out_pad = _k4(agg, dis2, gate, x_pad)
    return out_pad[:N, :D]
